# Initial kernel scaffold; baseline (speedup 1.0000x reference)
import jax
import jax.numpy as jnp
from jax.experimental import pallas as pl


def kernel(x, camera2ego, lidar2ego, camera_intrinsics, camera2lidar, img_aug_matrix, lidar_aug_matrix):
    raise NotImplementedError("write your pallas kernel here")


# speedup vs baseline: 3.7465x; 3.7465x over previous
import functools

import jax
import jax.numpy as jnp
import numpy as np
from jax import lax
from jax.experimental import pallas as pl
from jax.experimental.pallas import tpu as pltpu
from jax.experimental.pallas import tpu_sc as plsc

IH, IW = 256, 704
FH, FW = 16, 44
D_BINS = 59
N_CAM = 6
C_OUT = 80
C_HALF = 40
NX0, NX1 = 180, 180
NPTS = N_CAM * D_BINS * FH * FW
PT_ROWS = NPTS // 128
PTS_PER_CAM = D_BINS * FH * FW
CAM_ROWS = 328

GRID = NX0 * NX1
HALF = GRID // 2
DUMMY_SPREAD = 1024
ACC_ROWS = 17280
N_SC = 2
N_TILE = 16

_LO0 = float(np.float32(-53.7) - np.float32(0.6) / np.float32(2.0))
_LO2 = float(np.float32(0.0) - np.float32(20.0) / np.float32(2.0))
_DX0 = float(np.float32(0.6))
_DX2 = float(np.float32(20.0))


def _inv3(a):
    c00 = a[..., 1, 1] * a[..., 2, 2] - a[..., 1, 2] * a[..., 2, 1]
    c01 = a[..., 1, 2] * a[..., 2, 0] - a[..., 1, 0] * a[..., 2, 2]
    c02 = a[..., 1, 0] * a[..., 2, 1] - a[..., 1, 1] * a[..., 2, 0]
    c10 = a[..., 0, 2] * a[..., 2, 1] - a[..., 0, 1] * a[..., 2, 2]
    c11 = a[..., 0, 0] * a[..., 2, 2] - a[..., 0, 2] * a[..., 2, 0]
    c12 = a[..., 0, 1] * a[..., 2, 0] - a[..., 0, 0] * a[..., 2, 1]
    c20 = a[..., 0, 1] * a[..., 1, 2] - a[..., 0, 2] * a[..., 1, 1]
    c21 = a[..., 0, 2] * a[..., 1, 0] - a[..., 0, 0] * a[..., 1, 2]
    c22 = a[..., 0, 0] * a[..., 1, 1] - a[..., 0, 1] * a[..., 1, 0]
    det = a[..., 0, 0] * c00 + a[..., 0, 1] * c01 + a[..., 0, 2] * c02
    adj = jnp.stack([
        jnp.stack([c00, c10, c20], axis=-1),
        jnp.stack([c01, c11, c21], axis=-1),
        jnp.stack([c02, c12, c22], axis=-1),
    ], axis=-2)
    return adj / det[..., None, None]


def _geom_idx_kernel(fx_ref, fy_ref, fd_ref, coef_ref, idx0_ref, idx1_ref):
    n = pl.program_id(0)
    fx = fx_ref[...]
    fy = fy_ref[...]
    fd = fd_ref[...]

    def sel(k):
        return coef_ref[n, k]

    def bfr(v):
        return v.astype(jnp.bfloat16).astype(jnp.float32)

    ax = bfr(fx - sel(0))
    ay = bfr(fy - sel(1))
    az = bfr(fd - sel(2))
    m = [bfr(jnp.full(fx.shape, sel(k), jnp.float32)) for k in range(3, 12)]
    p0 = m[0] * ax + m[1] * ay + m[2] * az
    p1 = m[3] * ax + m[4] * ay + m[5] * az
    p2 = m[6] * ax + m[7] * ay + m[8] * az
    q0 = bfr(p0 * p2)
    q1 = bfr(p1 * p2)
    q2 = bfr(p2)
    m = [bfr(jnp.full(fx.shape, sel(k), jnp.float32)) for k in range(12, 21)]
    r0 = m[0] * q0 + m[1] * q1 + m[2] * q2
    r1 = m[3] * q0 + m[4] * q1 + m[5] * q2
    r2 = m[6] * q0 + m[7] * q1 + m[8] * q2
    r0 = bfr(r0 + sel(21))
    r1 = bfr(r1 + sel(22))
    r2 = bfr(r2 + sel(23))
    m = [bfr(jnp.full(fx.shape, sel(k), jnp.float32)) for k in range(24, 33)]
    s0 = m[0] * r0 + m[1] * r1 + m[2] * r2
    s1 = m[3] * r0 + m[4] * r1 + m[5] * r2
    s2 = m[6] * r0 + m[7] * r1 + m[8] * r2
    s0 = s0 + sel(33)
    s1 = s1 + sel(34)
    s2 = s2 + sel(35)
    g0 = ((s0 - _LO0) / _DX0).astype(jnp.int32)
    g1 = ((s1 - _LO0) / _DX0).astype(jnp.int32)
    g2 = ((s2 - _LO2) / _DX2).astype(jnp.int32)
    kept = ((g0 >= 0) & (g0 < NX0) & (g1 >= 0) & (g1 < NX1)
            & (g2 >= 0) & (g2 < 1))
    flat = g0 * NX1 + g1
    rows = lax.broadcasted_iota(jnp.int32, fx.shape, 0)
    cols = lax.broadcasted_iota(jnp.int32, fx.shape, 1)
    pid = (n * CAM_ROWS + rows) * 128 + cols
    dummy = HALF + (pid & (DUMMY_SPREAD - 1))
    idx0_ref[...] = jnp.where(kept & (flat < HALF), flat, dummy)
    idx1_ref[...] = jnp.where(kept & (flat >= HALF), flat - HALF, dummy)


def _compute_idx(coeffs, frustum_cam):
    fx, fy, fd = (a.reshape(CAM_ROWS, 128) for a in frustum_cam)
    idx0, idx1 = pl.pallas_call(
        _geom_idx_kernel,
        grid=(N_CAM,),
        in_specs=[
            pl.BlockSpec((CAM_ROWS, 128), lambda n: (0, 0)),
            pl.BlockSpec((CAM_ROWS, 128), lambda n: (0, 0)),
            pl.BlockSpec((CAM_ROWS, 128), lambda n: (0, 0)),
            pl.BlockSpec(memory_space=pltpu.SMEM),
        ],
        out_specs=[
            pl.BlockSpec((CAM_ROWS, 128), lambda n: (n, 0)),
            pl.BlockSpec((CAM_ROWS, 128), lambda n: (n, 0)),
        ],
        out_shape=[
            jax.ShapeDtypeStruct((N_CAM * CAM_ROWS, 128), jnp.int32),
            jax.ShapeDtypeStruct((N_CAM * CAM_ROWS, 128), jnp.int32),
        ],
    )(fx, fy, fd, coeffs)

    def compact(i):
        return i.reshape(N_CAM, CAM_ROWS * 128)[:, :PTS_PER_CAM].reshape(NPTS)

    return compact(idx0), compact(idx1)


def _sc_scatter(x2, idx_all, zeros_acc):
    mesh = plsc.VectorSubcoreMesh(core_axis_name="c", subcore_axis_name="s")

    @functools.partial(
        pl.kernel,
        out_type=jax.ShapeDtypeStruct((GRID, C_OUT), jnp.float32),
        mesh=mesh,
        scratch_types=[
            pltpu.VMEM((128, C_OUT), jnp.float32),
            pltpu.VMEM((128, C_OUT), jnp.float32),
            pltpu.VMEM((128,), jnp.int32),
            pltpu.VMEM((128,), jnp.int32),
            pltpu.SemaphoreType.DMA,
            pltpu.SemaphoreType.DMA,
            pltpu.VMEM_SHARED((ACC_ROWS, C_OUT), jnp.float32),
        ],
        compiler_params=pltpu.CompilerParams(use_tc_tiling_on_sc=False),
    )
    def scatter_kernel(x_hbm, i_hbm, z_hbm, out_hbm,
                       xbuf0, xbuf1, ibuf0, ibuf1, sem0, sem1, accum):
        c = lax.axis_index("c")
        s = lax.axis_index("s")
        ioff = c * NPTS

        zc = ACC_ROWS // N_TILE
        pltpu.sync_copy(z_hbm.at[pl.ds(s * zc, zc)],
                        accum.at[pl.ds(s * zc, zc)])
        plsc.subcore_barrier()

        def loads(k, xb, ib, sem):
            w = s + N_TILE * k
            return (
                pltpu.make_async_copy(
                    i_hbm.at[pl.ds(ioff + w * 128, 128)], ib, sem),
                pltpu.make_async_copy(x_hbm.at[pl.ds(w * 128, 128)], xb, sem),
            )

        def start(k, xb, ib, sem):
            a, b = loads(k, xb, ib, sem)
            a.start()
            b.start()

        def finish(k, xb, ib, sem):
            a, b = loads(k, xb, ib, sem)
            a.wait()
            b.wait()

        start(0, xbuf0, ibuf0, sem0)

        def body(t, _):
            k0 = 2 * t
            k1 = 2 * t + 1
            k2 = 2 * t + 2

            @pl.when(s + N_TILE * k1 < PT_ROWS)
            def _():
                start(k1, xbuf1, ibuf1, sem1)

            finish(k0, xbuf0, ibuf0, sem0)
            pltpu.sync_copy(xbuf0, accum.at[ibuf0], add=True)

            @pl.when(s + N_TILE * k2 < PT_ROWS)
            def _():
                start(k2, xbuf0, ibuf0, sem0)

            @pl.when(s + N_TILE * k1 < PT_ROWS)
            def _():
                finish(k1, xbuf1, ibuf1, sem1)
                pltpu.sync_copy(xbuf1, accum.at[ibuf1], add=True)

            return 0

        lax.fori_loop(0, 61, body, 0)
        plsc.subcore_barrier()

        cw = 200
        nw = HALF // cw
        for k in range((nw + N_TILE - 1) // N_TILE):
            w = s + N_TILE * k

            @pl.when(w < nw)
            def _():
                pltpu.sync_copy(accum.at[pl.ds(w * cw, cw)],
                                out_hbm.at[pl.ds(c * HALF + w * cw, cw)])

    return scatter_kernel(x2, idx_all, zeros_acc)


def kernel(x, camera2ego, lidar2ego, camera_intrinsics, camera2lidar,
           img_aug_matrix, lidar_aug_matrix):
    B = x.shape[0]
    assert B == 1 and x.shape[1] == N_CAM

    intrins = camera_intrinsics[..., :3, :3]
    post_rots = img_aug_matrix[..., :3, :3]
    post_trans = img_aug_matrix[..., :3, 3]
    c2l_rots = camera2lidar[..., :3, :3]
    c2l_trans = camera2lidar[..., :3, 3]
    extra_rots = lidar_aug_matrix[..., :3, :3]
    extra_trans = lidar_aug_matrix[..., :3, 3]
    inv_pr = _inv3(post_rots)
    combine = jnp.einsum('bnij,bnjk->bnik', c2l_rots, _inv3(intrins))

    coeffs = jnp.concatenate([
        post_trans[0],
        inv_pr[0].reshape(N_CAM, 9),
        combine[0].reshape(N_CAM, 9),
        c2l_trans[0],
        jnp.broadcast_to(extra_rots[0].reshape(1, 9), (N_CAM, 9)),
        jnp.broadcast_to(extra_trans[0].reshape(1, 3), (N_CAM, 3)),
        jnp.zeros((N_CAM, 4), jnp.float32),
    ], axis=1)

    ds = jnp.arange(1.0, 60.0, 1.0, dtype=jnp.float32)
    ds = jnp.broadcast_to(ds.reshape(-1, 1, 1), (D_BINS, FH, FW))
    xs_c = jnp.broadcast_to(
        jnp.linspace(0.0, IW - 1.0, FW, dtype=jnp.float32).reshape(1, 1, FW),
        (D_BINS, FH, FW))
    ys_c = jnp.broadcast_to(
        jnp.linspace(0.0, IH - 1.0, FH, dtype=jnp.float32).reshape(1, FH, 1),
        (D_BINS, FH, FW))
    pad = CAM_ROWS * 128 - PTS_PER_CAM

    def flat_pad(a):
        a = jnp.transpose(a, (0, 2, 1)).reshape(-1)
        return jnp.concatenate([a, jnp.zeros((pad,), jnp.float32)])

    frustum_cam = (flat_pad(xs_c), flat_pad(ys_c), flat_pad(ds))

    idx0, idx1 = _compute_idx(coeffs, frustum_cam)
    idx_all = jnp.concatenate([idx0, idx1])
    x2 = x.transpose(0, 1, 2, 4, 3, 5).reshape(NPTS, C_OUT)
    zeros_acc = jnp.zeros((ACC_ROWS, C_OUT), jnp.float32)
    out = _sc_scatter(x2, idx_all, zeros_acc)
    return jnp.transpose(out, (1, 0)).reshape(1, C_OUT, NX0, NX1)
